# baseline (speedup 1.0000x reference)
import jax
import jax.numpy as jnp
from jax.experimental import pallas as pl
from jax.experimental.pallas import tpu as pltpu

_M = 8192
_K = 2048
_E = 16
_BM = 1024


def _router_body(x_ref, w_ref, gate_ref, val_ref, idx_ref):
    x = x_ref[...]
    w = w_ref[...]
    lt = jax.lax.dot_general(
        w, x, (((1,), (1,)), ((), ())), preferred_element_type=jnp.float32
    )
    m = jnp.max(lt, axis=0, keepdims=True)
    e = jnp.exp(lt - m)
    s = jnp.sum(e, axis=0, keepdims=True)
    gt = e / s
    gate_ref[...] = gt

    lanef = jax.lax.broadcasted_iota(jnp.int32, gt.shape, 0).astype(jnp.float32)
    v1 = jnp.max(gt, axis=0, keepdims=True)
    i1 = jnp.min(jnp.where(gt == v1, lanef, 16.0), axis=0, keepdims=True)
    g2 = jnp.where(lanef == i1, -1.0, gt)
    v2 = jnp.max(g2, axis=0, keepdims=True)
    i2 = jnp.min(jnp.where(g2 == v2, lanef, 16.0), axis=0, keepdims=True)

    val_ref[...] = jnp.concatenate([v1, v2], axis=0)
    idx_ref[...] = jnp.concatenate([i1, i2], axis=0).astype(jnp.int32)


@jax.jit
def kernel(x, W):
    grid = (_M // _BM,)
    gate_t, val_t, idx_t = pl.pallas_call(
        _router_body,
        grid=grid,
        in_specs=[
            pl.BlockSpec((_BM, _K), lambda i: (i, 0)),
            pl.BlockSpec((_E, _K), lambda i: (0, 0)),
        ],
        out_specs=[
            pl.BlockSpec((_E, _BM), lambda i: (0, i)),
            pl.BlockSpec((2, _BM), lambda i: (0, i)),
            pl.BlockSpec((2, _BM), lambda i: (0, i)),
        ],
        out_shape=[
            jax.ShapeDtypeStruct((_E, _M), jnp.float32),
            jax.ShapeDtypeStruct((2, _M), jnp.float32),
            jax.ShapeDtypeStruct((2, _M), jnp.int32),
        ],
        compiler_params=pltpu.CompilerParams(
            dimension_semantics=("arbitrary",),
        ),
    )(x, W)
    return (val_t.T, idx_t.T, gate_t.T)
